# baseline (speedup 1.0000x reference)
import functools

import jax
import jax.numpy as jnp
from jax import lax
from jax.experimental import pallas as pl
from jax.experimental.pallas import tpu as pltpu
from jax.experimental.pallas import tpu_sc as plsc

D = 64
SCALE = 8.0
CB = 256


@functools.lru_cache(maxsize=None)
def _make_kernel(B, L, V):
    NC, NS = 2, 16
    NW = NC * NS
    n_bt = B // 128
    n_pr = B // CB
    n_units = L * n_pr
    assert n_units % NW == 0
    u_w = n_units // NW

    mesh = plsc.VectorSubcoreMesh(
        core_axis_name="c", subcore_axis_name="s", num_cores=NC, num_subcores=NS)

    @functools.partial(
        pl.kernel,
        out_type=jax.ShapeDtypeStruct((L, D // 8, n_bt, 8, 128), jnp.float32),
        mesh=mesh,
        scratch_types=[
            pltpu.VMEM((1, CB), jnp.int32),
            pltpu.VMEM((1, CB), jnp.int32),
            pltpu.VMEM((CB, D), jnp.float32),
            pltpu.VMEM((L, D), jnp.float32),
            pltpu.VMEM((D // 8, CB // 128, 8, 128), jnp.float32),
            pltpu.SemaphoreType.DMA,
        ],
        compiler_params=pltpu.CompilerParams(
            use_tc_tiling_on_sc=False, needs_layout_passes=False),
    )
    def body(w_hbm, p_hbm, emb0_hbm, emb1_hbm, out_hbm, wi_v, pi_v, wrow_v,
             emb1_v, obuf_v, sem):
        wid = lax.axis_index("s") * NC + lax.axis_index("c")
        u0 = wid * u_w

        pltpu.sync_copy(emb1_hbm, emb1_v)

        iota16 = lax.iota(jnp.int32, 16)

        def unit_body(t, carry):
            u = u0 + t
            l = u // n_pr
            b0 = pl.multiple_of((u % n_pr) * CB, CB)
            bt0 = pl.multiple_of((u % n_pr) * (CB // 128), CB // 128)
            pltpu.sync_copy(w_hbm.at[pl.ds(l, 1), pl.ds(b0, CB)], wi_v)
            pltpu.sync_copy(p_hbm.at[pl.ds(l, 1), pl.ds(b0, CB)], pi_v)
            waits = []
            for g in range(CB // 128):
                sl = pl.ds(g * 128, 128)
                waits.append(pltpu.async_copy(
                    emb0_hbm.at[wi_v.at[0, sl]], wrow_v.at[sl], sem))
            for w in waits:
                w.wait()

            def g_body(g, carry2):
                gt = g // 8
                gg = g % 8
                bvec = iota16 + g * 16
                pvec = pi_v[0, pl.ds(g * 16, 16)]

                def dt_body(dt, carry3):
                    for di in range(8):
                        d = dt * 8 + di
                        dvec = jnp.full((16,), d, jnp.int32)
                        wv = plsc.load_gather(wrow_v, [bvec, dvec])
                        pv = plsc.load_gather(emb1_v, [pvec, dvec])
                        obuf_v[dt, gt, di, pl.ds(gg * 16, 16)] = wv * SCALE + pv
                    return carry3

                lax.fori_loop(0, D // 8, dt_body, 0)
                return carry2

            lax.fori_loop(0, CB // 16, g_body, 0)

            for dt in range(D // 8):
                pltpu.sync_copy(obuf_v.at[dt],
                                out_hbm.at[l, dt, pl.ds(bt0, CB // 128)])
            return carry

        lax.fori_loop(0, u_w, unit_body, 0)

    return body


def kernel(src_word, src_pos, emb0_table, emb1_table):
    B, L, _ = src_word.shape
    V = emb0_table.shape[0]
    w2 = jnp.transpose(src_word.astype(jnp.int32), (1, 2, 0)).reshape(L, B)
    p2 = jnp.transpose(src_pos.astype(jnp.int32), (1, 2, 0)).reshape(L, B)
    out5 = _make_kernel(B, L, V)(w2, p2, emb0_table, emb1_table)
    out = jnp.transpose(out5, (2, 4, 0, 1, 3)).reshape(B, L, D)
    return out


# speedup vs baseline: 1.3131x; 1.3131x over previous
import functools

import jax
import jax.numpy as jnp
from jax import lax
from jax.experimental import pallas as pl
from jax.experimental.pallas import tpu as pltpu
from jax.experimental.pallas import tpu_sc as plsc

D = 64
SCALE = 8.0
CB = 256
PITCH = 64


@functools.lru_cache(maxsize=None)
def _make_kernel(B, L, V):
    NC, NS = 2, 16
    NW = NC * NS
    n_bt = B // 128
    n_pr = B // CB
    n_units = L * n_pr
    assert n_units % NW == 0
    u_w = n_units // NW

    mesh = plsc.VectorSubcoreMesh(
        core_axis_name="c", subcore_axis_name="s", num_cores=NC, num_subcores=NS)

    @functools.partial(
        pl.kernel,
        out_type=jax.ShapeDtypeStruct((L, D // 8, n_bt, 8, 128), jnp.float32),
        mesh=mesh,
        scratch_types=[
            pltpu.VMEM((1, CB), jnp.int32),
            pltpu.VMEM((1, CB), jnp.int32),
            pltpu.VMEM((CB, PITCH), jnp.float32),
            pltpu.VMEM((CB, PITCH), jnp.float32),
            pltpu.VMEM((D // 8, CB // 128, 8, 128), jnp.float32),
            pltpu.SemaphoreType.DMA,
        ],
        compiler_params=pltpu.CompilerParams(
            use_tc_tiling_on_sc=False, needs_layout_passes=False),
    )
    def body(w_hbm, p_hbm, emb0_hbm, emb1_hbm, out_hbm, wi_v, pi_v, wrow_v,
             prow_v, obuf_v, sem):
        wid = lax.axis_index("s") * NC + lax.axis_index("c")
        u0 = wid * u_w
        iota16 = lax.iota(jnp.int32, 16)

        def unit_body(t, carry):
            u = u0 + t
            l = u // n_pr
            b0 = pl.multiple_of((u % n_pr) * CB, CB)
            bt0 = pl.multiple_of((u % n_pr) * (CB // 128), CB // 128)
            pltpu.sync_copy(w_hbm.at[pl.ds(l, 1), pl.ds(b0, CB)], wi_v)
            pltpu.sync_copy(p_hbm.at[pl.ds(l, 1), pl.ds(b0, CB)], pi_v)
            waits = []
            for g in range(CB // 128):
                sl = pl.ds(g * 128, 128)
                waits.append(pltpu.async_copy(
                    emb0_hbm.at[wi_v.at[0, sl]], wrow_v.at[sl], sem))
                waits.append(pltpu.async_copy(
                    emb1_hbm.at[pi_v.at[0, sl]], prow_v.at[sl], sem))
            for w in waits:
                w.wait()

            @plsc.parallel_loop(0, CB // 16)
            def g_body(g):
                gt = g // 8
                gg = g % 8
                bvec = iota16 + g * 16
                for d in range(D):
                    dvec = jnp.full((16,), d, jnp.int32)
                    wv = plsc.load_gather(wrow_v, [bvec, dvec])
                    pv = plsc.load_gather(prow_v, [bvec, dvec])
                    obuf_v[d // 8, gt, d % 8, pl.ds(gg * 16, 16)] = (
                        wv * SCALE + pv)

            pltpu.sync_copy(obuf_v,
                            out_hbm.at[l, :, pl.ds(bt0, CB // 128)])
            return carry

        lax.fori_loop(0, u_w, unit_body, 0)

    return body


def kernel(src_word, src_pos, emb0_table, emb1_table):
    B, L, _ = src_word.shape
    V = emb0_table.shape[0]
    w2 = jnp.transpose(src_word.astype(jnp.int32), (1, 2, 0)).reshape(L, B)
    p2 = jnp.transpose(src_pos.astype(jnp.int32), (1, 2, 0)).reshape(L, B)
    out5 = _make_kernel(B, L, V)(w2, p2, emb0_table, emb1_table)
    out = jnp.transpose(out5, (2, 4, 0, 1, 3)).reshape(B, L, D)
    return out


# speedup vs baseline: 1.9649x; 1.4964x over previous
import functools

import jax
import jax.numpy as jnp
from jax import lax
from jax.experimental import pallas as pl
from jax.experimental.pallas import tpu as pltpu
from jax.experimental.pallas import tpu_sc as plsc

D = 64
SCALE = 8.0
CB = 256


@functools.lru_cache(maxsize=None)
def _make_kernel(B, L, V):
    NC, NS = 2, 16
    NW = NC * NS
    n_pr = B // CB
    n_units = L * n_pr
    assert n_units % (2 * NW) == 0
    u_w = n_units // NW

    mesh = plsc.VectorSubcoreMesh(
        core_axis_name="c", subcore_axis_name="s", num_cores=NC, num_subcores=NS)

    buf_set = [
        pltpu.VMEM((1, CB), jnp.int32),
        pltpu.VMEM((1, CB), jnp.int32),
        pltpu.VMEM((CB, D), jnp.float32),
        pltpu.VMEM((CB, D), jnp.float32),
        pltpu.VMEM((CB, D), jnp.float32),
        pltpu.SemaphoreType.DMA,
        pltpu.SemaphoreType.DMA,
        pltpu.SemaphoreType.DMA,
    ]

    @functools.partial(
        pl.kernel,
        out_type=jax.ShapeDtypeStruct((B, L * D), jnp.float32),
        mesh=mesh,
        scratch_types=buf_set + buf_set,
        compiler_params=pltpu.CompilerParams(
            use_tc_tiling_on_sc=False, needs_layout_passes=False),
    )
    def body(w_hbm, p_hbm, emb0_hbm, emb1_hbm, out_hbm,
             wiA, piA, wrowA, prowA, obufA, semIA, semGA, semOA,
             wiB, piB, wrowB, prowB, obufB, semIB, semGB, semOB):
        wid = lax.axis_index("s") * NC + lax.axis_index("c")
        u0 = wid * u_w

        def coords(u):
            l = u // n_pr
            b0 = pl.multiple_of((u % n_pr) * CB, CB)
            return l, b0

        def stage_idx(u, wi, pi, sem):
            l, b0 = coords(u)
            pltpu.async_copy(w_hbm.at[pl.ds(l, 1), pl.ds(b0, CB)], wi, sem)
            pltpu.async_copy(p_hbm.at[pl.ds(l, 1), pl.ds(b0, CB)], pi, sem)

        def wait_idx(wi, pi, sem):
            pltpu.make_async_copy(w_hbm.at[pl.ds(0, 1), pl.ds(0, CB)], wi, sem).wait()
            pltpu.make_async_copy(p_hbm.at[pl.ds(0, 1), pl.ds(0, CB)], pi, sem).wait()

        def fire_gathers(wi, pi, wrow, prow, sem):
            for g in range(CB // 128):
                sl = pl.ds(g * 128, 128)
                pltpu.async_copy(emb0_hbm.at[wi.at[0, sl]], wrow.at[sl], sem)
                pltpu.async_copy(emb1_hbm.at[pi.at[0, sl]], prow.at[sl], sem)

        def wait_gathers(wi, pi, wrow, prow, sem):
            for g in range(CB // 128):
                sl = pl.ds(g * 128, 128)
                pltpu.make_async_copy(emb0_hbm.at[wi.at[0, sl]], wrow.at[sl], sem).wait()
                pltpu.make_async_copy(emb1_hbm.at[pi.at[0, sl]], prow.at[sl], sem).wait()

        def compute(wrow, prow, obuf):
            @plsc.parallel_loop(0, CB, unroll=2)
            def r_body(r):
                for c in range(D // 16):
                    sl = pl.ds(c * 16, 16)
                    obuf[r, sl] = wrow[r, sl] * SCALE + prow[r, sl]

        def fire_out(u, obuf, sem):
            l, b0 = coords(u)
            loff = pl.multiple_of(l * D, D)
            pltpu.async_copy(obuf, out_hbm.at[pl.ds(b0, CB), pl.ds(loff, D)], sem)

        def wait_out(obuf, sem):
            pltpu.make_async_copy(
                obuf, out_hbm.at[pl.ds(0, CB), pl.ds(0, D)], sem).wait()

        stage_idx(u0, wiA, piA, semIA)
        wait_idx(wiA, piA, semIA)
        fire_gathers(wiA, piA, wrowA, prowA, semGA)
        stage_idx(u0 + 1, wiB, piB, semIB)

        def pair_body(j, carry):
            uA = u0 + 2 * j
            uB = uA + 1
            wait_idx(wiB, piB, semIB)
            fire_gathers(wiB, piB, wrowB, prowB, semGB)
            wait_gathers(wiA, piA, wrowA, prowA, semGA)

            @pl.when(j > 0)
            def _():
                wait_out(obufA, semOA)

            compute(wrowA, prowA, obufA)
            fire_out(uA, obufA, semOA)

            @pl.when(j < u_w // 2 - 1)
            def _():
                stage_idx(uA + 2, wiA, piA, semIA)

            wait_gathers(wiB, piB, wrowB, prowB, semGB)

            @pl.when(j > 0)
            def _():
                wait_out(obufB, semOB)

            compute(wrowB, prowB, obufB)
            fire_out(uB, obufB, semOB)

            @pl.when(j < u_w // 2 - 1)
            def _():
                wait_idx(wiA, piA, semIA)
                fire_gathers(wiA, piA, wrowA, prowA, semGA)
                stage_idx(uB + 2, wiB, piB, semIB)

            return carry

        lax.fori_loop(0, u_w // 2, pair_body, 0)
        wait_out(obufA, semOA)
        wait_out(obufB, semOB)

    return body


def kernel(src_word, src_pos, emb0_table, emb1_table):
    B, L, _ = src_word.shape
    V = emb0_table.shape[0]
    w2 = jnp.transpose(src_word.astype(jnp.int32), (1, 2, 0)).reshape(L, B)
    p2 = jnp.transpose(src_pos.astype(jnp.int32), (1, 2, 0)).reshape(L, B)
    out = _make_kernel(B, L, V)(w2, p2, emb0_table, emb1_table)
    return out.reshape(B, L, D)


# speedup vs baseline: 2.5853x; 1.3157x over previous
import functools

import jax
import jax.numpy as jnp
from jax import lax
from jax.experimental import pallas as pl
from jax.experimental.pallas import tpu as pltpu
from jax.experimental.pallas import tpu_sc as plsc

D = 64
SCALE = 8.0
CB = 256


@functools.lru_cache(maxsize=None)
def _make_kernel(B, L, V):
    NC, NS = 2, 16
    NW = NC * NS
    n_pr = B // CB
    n_units = L * n_pr
    assert n_units % (2 * NW) == 0
    u_w = n_units // NW

    mesh = plsc.VectorSubcoreMesh(
        core_axis_name="c", subcore_axis_name="s", num_cores=NC, num_subcores=NS)

    buf_set = [
        pltpu.VMEM((1, CB), jnp.int32),
        pltpu.VMEM((1, CB), jnp.int32),
        pltpu.VMEM((CB, D), jnp.float32),
        pltpu.VMEM((CB, D), jnp.float32),
        pltpu.SemaphoreType.DMA,
        pltpu.SemaphoreType.DMA,
        pltpu.SemaphoreType.DMA,
    ]

    @functools.partial(
        pl.kernel,
        out_type=jax.ShapeDtypeStruct((B, L * D), jnp.float32),
        mesh=mesh,
        scratch_types=buf_set + buf_set + [pltpu.VMEM((L, D), jnp.float32)],
        compiler_params=pltpu.CompilerParams(
            use_tc_tiling_on_sc=False, needs_layout_passes=False),
    )
    def body(w_hbm, p_hbm, emb0_hbm, emb1_hbm, out_hbm,
             wiA, piA, wrowA, obufA, semIA, semGA, semOA,
             wiB, piB, wrowB, obufB, semIB, semGB, semOB, emb1_v):
        wid = lax.axis_index("s") * NC + lax.axis_index("c")
        u0 = wid * u_w
        pltpu.sync_copy(emb1_hbm, emb1_v)

        def coords(u):
            l = u // n_pr
            b0 = pl.multiple_of((u % n_pr) * CB, CB)
            return l, b0

        def stage_idx(u, wi, pi, sem):
            l, b0 = coords(u)
            pltpu.async_copy(w_hbm.at[pl.ds(l, 1), pl.ds(b0, CB)], wi, sem)
            pltpu.async_copy(p_hbm.at[pl.ds(l, 1), pl.ds(b0, CB)], pi, sem)

        def wait_idx(wi, pi, sem):
            pltpu.make_async_copy(w_hbm.at[pl.ds(0, 1), pl.ds(0, CB)], wi, sem).wait()
            pltpu.make_async_copy(p_hbm.at[pl.ds(0, 1), pl.ds(0, CB)], pi, sem).wait()

        def fire_gathers(wi, pi, wrow, sem):
            for g in range(CB // 128):
                sl = pl.ds(g * 128, 128)
                pltpu.async_copy(emb0_hbm.at[wi.at[0, sl]], wrow.at[sl], sem)

        def wait_gathers(wi, pi, wrow, sem):
            for g in range(CB // 128):
                sl = pl.ds(g * 128, 128)
                pltpu.make_async_copy(emb0_hbm.at[wi.at[0, sl]], wrow.at[sl], sem).wait()

        def compute(wrow, pi, obuf):
            @plsc.parallel_loop(0, CB // 16)
            def r_body(r16):
                r0 = pl.multiple_of(r16 * 16, 16)
                pvec = pi[0, pl.ds(r0, 16)]
                for i in range(16):
                    p = pvec[i]
                    for c in range(D // 16):
                        sl = pl.ds(c * 16, 16)
                        obuf[r0 + i, sl] = wrow[r0 + i, sl] * SCALE + emb1_v[p, sl]

        def fire_out(u, obuf, sem):
            l, b0 = coords(u)
            loff = pl.multiple_of(l * D, D)
            pltpu.async_copy(obuf, out_hbm.at[pl.ds(b0, CB), pl.ds(loff, D)], sem)

        def wait_out(obuf, sem):
            pltpu.make_async_copy(
                obuf, out_hbm.at[pl.ds(0, CB), pl.ds(0, D)], sem).wait()

        stage_idx(u0, wiA, piA, semIA)
        wait_idx(wiA, piA, semIA)
        fire_gathers(wiA, piA, wrowA, semGA)
        stage_idx(u0 + 1, wiB, piB, semIB)

        def pair_body(j, carry):
            uA = u0 + 2 * j
            uB = uA + 1
            wait_idx(wiB, piB, semIB)
            fire_gathers(wiB, piB, wrowB, semGB)
            wait_gathers(wiA, piA, wrowA, semGA)

            @pl.when(j > 0)
            def _():
                wait_out(obufA, semOA)

            compute(wrowA, piA, obufA)
            fire_out(uA, obufA, semOA)

            @pl.when(j < u_w // 2 - 1)
            def _():
                stage_idx(uA + 2, wiA, piA, semIA)

            wait_gathers(wiB, piB, wrowB, semGB)

            @pl.when(j > 0)
            def _():
                wait_out(obufB, semOB)

            compute(wrowB, piB, obufB)
            fire_out(uB, obufB, semOB)

            @pl.when(j < u_w // 2 - 1)
            def _():
                wait_idx(wiA, piA, semIA)
                fire_gathers(wiA, piA, wrowA, semGA)
                stage_idx(uB + 2, wiB, piB, semIB)

            return carry

        lax.fori_loop(0, u_w // 2, pair_body, 0)
        wait_out(obufA, semOA)
        wait_out(obufB, semOB)

    return body


def kernel(src_word, src_pos, emb0_table, emb1_table):
    B, L, _ = src_word.shape
    V = emb0_table.shape[0]
    w2 = jnp.transpose(src_word.astype(jnp.int32), (1, 2, 0)).reshape(L, B)
    p2 = jnp.transpose(src_pos.astype(jnp.int32), (1, 2, 0)).reshape(L, B)
    out = _make_kernel(B, L, V)(w2, p2, emb0_table, emb1_table)
    return out.reshape(B, L, D)
